# Initial kernel scaffold; baseline (speedup 1.0000x reference)
import jax
import jax.numpy as jnp
from jax.experimental import pallas as pl


def kernel(x):
    raise NotImplementedError("write your pallas kernel here")


# speedup vs baseline: 2.1776x; 2.1776x over previous
import functools

import jax
import jax.numpy as jnp
import numpy as np
from jax import lax
from jax.experimental import pallas as pl
from jax.experimental.pallas import tpu as pltpu
from jax.experimental.pallas import tpu_sc as plsc

_B, _C, _H, _W, _G = 8, 96, 224, 224, 4
_BH = _H // _G
_ROWS = _B * _C * _H * _G
_NC, _NS = 2, 16
_NW = _NC * _NS
_RPT = _ROWS // _NW
_CHUNK = 128
_NCHUNK = _RPT // _CHUNK


def _gather_idx() -> np.ndarray:
    rng = np.random.RandomState(0)
    perms = np.stack([rng.permutation(_G * _G) for _ in range(_B)], axis=0)
    inv = np.argsort(perms, axis=1)

    b = np.arange(_B)[:, None, None, None]
    c = np.arange(_C)[None, :, None, None]
    h = np.arange(_H)[None, None, :, None]
    tw = np.arange(_G)[None, None, None, :]
    th = h // _BH
    i = h % _BH
    src = inv[b, th * _G + tw]
    sh, sw = src // _G, src % _G
    gidx = ((b * _C + c) * _H + (sh * _BH + i)) * _G + sw
    return gidx.reshape(-1).astype(np.int32)


_GIDX = _gather_idx()


def _sc_body(x_hbm, gidx_hbm, out_hbm, idx_v, rows_v, sem):
    wid = lax.axis_index("s") * _NC + lax.axis_index("c")
    base = wid * _RPT

    def step(g, carry):
        off = base + g * _CHUNK
        pltpu.sync_copy(gidx_hbm.at[pl.ds(off, _CHUNK)], idx_v)
        pltpu.async_copy(x_hbm.at[idx_v], rows_v, sem).wait()
        pltpu.sync_copy(rows_v, out_hbm.at[pl.ds(off, _CHUNK)])
        return carry

    lax.fori_loop(0, _NCHUNK, step, 0)


_sc_call = pl.kernel(
    _sc_body,
    out_type=jax.ShapeDtypeStruct((_ROWS, _BH), jnp.float32),
    mesh=plsc.VectorSubcoreMesh(core_axis_name="c", subcore_axis_name="s"),
    scratch_types=[
        pltpu.VMEM((_CHUNK,), jnp.int32),
        pltpu.VMEM((_CHUNK, _BH), jnp.float32),
        pltpu.SemaphoreType.DMA,
    ],
    compiler_params=pltpu.CompilerParams(use_tc_tiling_on_sc=False),
)


def kernel(x):
    x2 = x.reshape(_ROWS, _BH)
    out2 = _sc_call(x2, jnp.asarray(_GIDX))
    return out2.reshape(_B, _C, _H, _W)


# speedup vs baseline: 3.0944x; 1.4210x over previous
import jax
import jax.numpy as jnp
import numpy as np
from jax import lax
from jax.experimental import pallas as pl
from jax.experimental.pallas import tpu as pltpu
from jax.experimental.pallas import tpu_sc as plsc

_B, _C, _H, _W, _G = 8, 96, 224, 224, 4
_BH = _H // _G
_ROWS = _B * _C * _H * _G
_NC, _NS = 2, 16
_NW = _NC * _NS
_RPT = _ROWS // _NW
_CHUNK = 128
_NCHUNK = _RPT // _CHUNK
_NBUF = 8
_LAG = 4
_NSUPER = _NCHUNK // _NBUF


def _gather_idx() -> np.ndarray:
    rng = np.random.RandomState(0)
    perms = np.stack([rng.permutation(_G * _G) for _ in range(_B)], axis=0)
    inv = np.argsort(perms, axis=1)

    b = np.arange(_B)[:, None, None, None]
    c = np.arange(_C)[None, :, None, None]
    h = np.arange(_H)[None, None, :, None]
    tw = np.arange(_G)[None, None, None, :]
    th = h // _BH
    i = h % _BH
    src = inv[b, th * _G + tw]
    sh, sw = src // _G, src % _G
    gidx = ((b * _C + c) * _H + (sh * _BH + i)) * _G + sw
    return gidx.reshape(_NW * _NCHUNK, _CHUNK).astype(np.int32)


_GIDX = _gather_idx()


def _sc_body(x_hbm, gidx_hbm, out_hbm, idx_v, rows, gsem, wsem):
    wid = lax.axis_index("s") * _NC + lax.axis_index("c")
    base = wid * _RPT

    pltpu.sync_copy(gidx_hbm.at[pl.ds(wid * _NCHUNK, _NCHUNK), :], idx_v)

    def gather_start(g, j):
        pltpu.make_async_copy(x_hbm.at[idx_v.at[g]], rows[j], gsem[j]).start()

    def gather_wait(j):
        pltpu.make_async_copy(x_hbm.at[pl.ds(0, _CHUNK)], rows[j], gsem[j]).wait()

    def write_start(g, j):
        pltpu.make_async_copy(
            rows[j], out_hbm.at[pl.ds(base + g * _CHUNK, _CHUNK)], wsem[j]
        ).start()

    def write_wait(j):
        pltpu.make_async_copy(
            rows[j], out_hbm.at[pl.ds(0, _CHUNK)], wsem[j]
        ).wait()

    def superstep(s, carry):
        for j in range(_NBUF):
            g = s * _NBUF + j
            if j < _LAG:

                @pl.when(s >= 1)
                def _():
                    write_wait(j)
                    gather_start(g, j)

                @pl.when(s < 1)
                def _():
                    gather_start(g, j)
            else:
                write_wait(j)
                gather_start(g, j)

            jd = (j - _LAG) % _NBUF
            gd = g - _LAG
            if j >= _LAG:
                gather_wait(jd)
                write_start(gd, jd)
            else:

                @pl.when(s >= 1)
                def _():
                    gather_wait(jd)
                    write_start(gd, jd)

        return carry

    def superstep0():
        for j in range(_NBUF):
            gather_start(j, j)
            if j >= _LAG:
                jd = j - _LAG
                gather_wait(jd)
                write_start(jd, jd)

    superstep0()
    lax.fori_loop(1, _NSUPER, superstep, 0)

    last = (_NSUPER - 1) * _NBUF
    for j in range(_LAG, _NBUF):
        gather_wait(j)
        write_start(last + j, j)
    for j in range(_NBUF):
        write_wait(j)


_sc_call = pl.kernel(
    _sc_body,
    out_type=jax.ShapeDtypeStruct((_ROWS, _BH), jnp.float32),
    mesh=plsc.VectorSubcoreMesh(core_axis_name="c", subcore_axis_name="s"),
    scratch_types=[
        pltpu.VMEM((_NCHUNK, _CHUNK), jnp.int32),
        [pltpu.VMEM((_CHUNK, _BH), jnp.float32) for _ in range(_NBUF)],
        [pltpu.SemaphoreType.DMA for _ in range(_NBUF)],
        [pltpu.SemaphoreType.DMA for _ in range(_NBUF)],
    ],
    compiler_params=pltpu.CompilerParams(use_tc_tiling_on_sc=False),
)


def kernel(x):
    x2 = x.reshape(_ROWS, _BH)
    out2 = _sc_call(x2, jnp.asarray(_GIDX))
    return out2.reshape(_B, _C, _H, _W)
